# Initial kernel scaffold; baseline (speedup 1.0000x reference)
import jax
import jax.numpy as jnp
from jax.experimental import pallas as pl


def kernel(x, W):
    raise NotImplementedError("write your pallas kernel here")


# speedup vs baseline: 2.9724x; 2.9724x over previous
import functools

import jax
import jax.numpy as jnp
from jax import lax
from jax.experimental import pallas as pl
from jax.experimental.pallas import tpu as pltpu
from jax.experimental.pallas import tpu_sc as plsc

D_MODEL = 128
CHUNK = 128


@functools.lru_cache(maxsize=None)
def _make_gather(B: int, V: int, D: int):
    info = plsc.get_sparse_core_info()
    nw = info.num_cores * info.num_subcores
    b_per_w = B // nw
    n_chunks = b_per_w // CHUNK

    @functools.partial(
        pl.kernel,
        mesh=plsc.VectorSubcoreMesh(core_axis_name="c", subcore_axis_name="s"),
        out_type=jax.ShapeDtypeStruct((B, D), jnp.float32),
        scratch_types=[
            pltpu.VMEM((n_chunks, CHUNK), jnp.int32),
            pltpu.VMEM((CHUNK, D), jnp.float32),
            pltpu.SemaphoreType.DMA,
        ],
    )
    def gather_kernel(idx_hbm, table_hbm, out_hbm, idx_v, rows_v, sem):
        wid = lax.axis_index("s") * info.num_cores + lax.axis_index("c")
        pltpu.sync_copy(idx_hbm.at[wid], idx_v)
        base = wid * b_per_w

        def body(j, carry):
            pltpu.async_copy(table_hbm.at[idx_v.at[j]], rows_v, sem).wait()
            pltpu.sync_copy(rows_v, out_hbm.at[pl.ds(base + j * CHUNK, CHUNK)])
            return carry

        lax.fori_loop(0, n_chunks, body, 0)

    return gather_kernel


def kernel(x, W):
    B = x.shape[0] * x.shape[1]
    fn = _make_gather(B, W.shape[0], W.shape[1])
    info = plsc.get_sparse_core_info()
    nw = info.num_cores * info.num_subcores
    idx = x.astype(jnp.int32).reshape(nw, (B // nw) // CHUNK, CHUNK)
    out = fn(idx, W)
    return out.reshape(x.shape[0], x.shape[1], W.shape[1])


# speedup vs baseline: 3.3408x; 1.1240x over previous
import functools

import jax
import jax.numpy as jnp
from jax import lax
from jax.experimental import pallas as pl
from jax.experimental.pallas import tpu as pltpu
from jax.experimental.pallas import tpu_sc as plsc

D_MODEL = 128
CHUNK = 128
NBUF = 5


@functools.lru_cache(maxsize=None)
def _make_gather(B: int, V: int, D: int):
    info = plsc.get_sparse_core_info()
    nw = info.num_cores * info.num_subcores
    b_per_w = B // nw
    n = b_per_w // CHUNK
    lead = NBUF - 1
    n_outer = n // NBUF

    @functools.partial(
        pl.kernel,
        mesh=plsc.VectorSubcoreMesh(core_axis_name="c", subcore_axis_name="s"),
        out_type=jax.ShapeDtypeStruct((B, D), jnp.float32),
        scratch_types=(
            [pltpu.VMEM((n, CHUNK), jnp.int32),
             pltpu.VMEM((NBUF, CHUNK, D), jnp.float32)]
            + [pltpu.SemaphoreType.DMA] * (2 * NBUF)
        ),
    )
    def gather_kernel(idx_hbm, table_hbm, out_hbm, idx_v, rows_v, *sems):
        gsem = sems[:NBUF]
        ssem = sems[NBUF:]
        wid = lax.axis_index("s") * info.num_cores + lax.axis_index("c")
        pltpu.sync_copy(idx_hbm.at[wid], idx_v)
        base = wid * b_per_w

        def gather_start(j, b):
            pltpu.async_copy(table_hbm.at[idx_v.at[j]], rows_v.at[b], gsem[b])

        def gather_wait(j, b):
            pltpu.make_async_copy(
                table_hbm.at[idx_v.at[j]], rows_v.at[b], gsem[b]).wait()

        def store_start(j, b):
            pltpu.async_copy(
                rows_v.at[b], out_hbm.at[pl.ds(base + j * CHUNK, CHUNK)],
                ssem[b])

        def store_wait(j, b):
            pltpu.make_async_copy(
                rows_v.at[b], out_hbm.at[pl.ds(base + j * CHUNK, CHUNK)],
                ssem[b]).wait()

        def step(j, b, first, last):
            bg = (b + lead) % NBUF
            if not last:
                if not first:
                    store_wait(j - 1, bg)
                gather_start(j + lead, bg)
            gather_wait(j, b)
            store_start(j, b)

        for m in range(lead):
            gather_start(m, m)
        for b in range(NBUF):
            step(b, b, first=(b == 0), last=False)

        def outer(j0, carry):
            for b in range(NBUF):
                step(j0 * NBUF + b, b, first=False, last=False)
            return carry

        lax.fori_loop(1, n_outer - 1, outer, 0)

        for b in range(NBUF):
            j = (n_outer - 1) * NBUF + b
            step(j, b, first=False, last=(j + lead >= n))
        for b in range(NBUF):
            store_wait(n - NBUF + b, b)

    return gather_kernel


def kernel(x, W):
    B = x.shape[0] * x.shape[1]
    fn = _make_gather(B, W.shape[0], W.shape[1])
    info = plsc.get_sparse_core_info()
    nw = info.num_cores * info.num_subcores
    idx = x.astype(jnp.int32).reshape(nw, (B // nw) // CHUNK, CHUNK)
    out = fn(idx, W)
    return out.reshape(x.shape[0], x.shape[1], W.shape[1])
